# baseline (speedup 1.0000x reference)
import jax
import jax.numpy as jnp
from jax import lax
from jax.experimental import pallas as pl
from jax.experimental.pallas import tpu as pltpu
from jax.experimental.pallas import tpu_sc as plsc

_L = 16
_TPW = 256


def _dense_body(x_ref, w_ref, b_ref, v0_ref, v1_ref, e0_ref, e1_ref,
                aux_ref, z_ref):
    g = pl.program_id(0)
    G = pl.num_programs(0)
    T = x_ref.shape[1]
    E = w_ref.shape[1]

    x = x_ref[0]
    w = w_ref[...]
    logits = jax.lax.dot_general(
        x, w, (((1,), (0,)), ((), ())),
        preferred_element_type=jnp.float32) + b_ref[...]

    m = jnp.max(logits, axis=-1, keepdims=True)
    ex = jnp.exp(logits - m)
    s = jnp.sum(ex, axis=-1, keepdims=True)
    probs = ex / s

    iota_e = jax.lax.broadcasted_iota(jnp.int32, (T, E), 1)
    v0 = jnp.max(probs, axis=-1, keepdims=True)
    e0 = jnp.min(jnp.where(probs == v0, iota_e, E), axis=-1, keepdims=True)
    probs1 = jnp.where(iota_e == e0, -jnp.inf, probs)
    v1 = jnp.max(probs1, axis=-1, keepdims=True)
    e1 = jnp.min(jnp.where(probs1 == v1, iota_e, E), axis=-1, keepdims=True)

    logz = m + jnp.log(s)
    z_part = jnp.sum(logz * logz) / (G * T)
    a0 = (iota_e == e0).astype(jnp.float32)
    a1 = (iota_e == e1).astype(jnp.float32)
    em_mean = jnp.sum(jnp.maximum(a0, a1), axis=0, keepdims=True) / T
    pm_mean = jnp.sum(probs, axis=0, keepdims=True) / T
    aux_part = jnp.sum(em_mean * pm_mean) * (E / G)

    @pl.when(g == 0)
    def _init():
        aux_ref[...] = jnp.zeros_like(aux_ref)
        z_ref[...] = jnp.zeros_like(z_ref)

    aux_ref[...] += jnp.reshape(aux_part, (1, 1))
    z_ref[...] += jnp.reshape(z_part, (1, 1))

    v0_ref[...] = v0[None]
    v1_ref[...] = v1[None]
    e0_ref[...] = e0[None]
    e1_ref[...] = e1[None]


def _build_dense(G, T, D, E, interpret=False):
    return pl.pallas_call(
        _dense_body,
        grid=(G,),
        in_specs=[
            pl.BlockSpec((1, T, D), lambda g: (g, 0, 0)),
            pl.BlockSpec((D, E), lambda g: (0, 0)),
            pl.BlockSpec((1, E), lambda g: (0, 0)),
        ],
        out_specs=[
            pl.BlockSpec((1, T, 1), lambda g: (g, 0, 0)),
            pl.BlockSpec((1, T, 1), lambda g: (g, 0, 0)),
            pl.BlockSpec((1, T, 1), lambda g: (g, 0, 0)),
            pl.BlockSpec((1, T, 1), lambda g: (g, 0, 0)),
            pl.BlockSpec((1, 1), lambda g: (0, 0)),
            pl.BlockSpec((1, 1), lambda g: (0, 0)),
        ],
        out_shape=[
            jax.ShapeDtypeStruct((G, T, 1), jnp.float32),
            jax.ShapeDtypeStruct((G, T, 1), jnp.float32),
            jax.ShapeDtypeStruct((G, T, 1), jnp.int32),
            jax.ShapeDtypeStruct((G, T, 1), jnp.int32),
            jax.ShapeDtypeStruct((1, 1), jnp.float32),
            jax.ShapeDtypeStruct((1, 1), jnp.float32),
        ],
        interpret=interpret,
    )


def _sc_router_body(v0h, v1h, e0h, e1h, caph,
                    p0h, p1h, cw0h, cw1h,
                    v0l, e0l, e1l, v1s, segv, segi, part0, part1,
                    idx2, zbuf, pb0, pb1, cb0, cb1, capl, sp0, sp1):
    c = lax.axis_index("c")
    s = lax.axis_index("s")
    T = v0l.shape[0]
    g = 2 * c + s // 8
    row = s // 8
    ebase = (s % 8) * 2
    tsl = (s % 8) * _TPW
    lanes = lax.iota(jnp.int32, _L)

    pltpu.sync_copy(v0h.at[g], v0l)
    pltpu.sync_copy(e0h.at[g], e0l)
    pltpu.sync_copy(e1h.at[g], e1l)
    pltpu.sync_copy(v1h.at[g, pl.ds(tsl, _TPW)], v1s)
    pltpu.sync_copy(caph, capl)

    zero16 = jnp.zeros((_L,), jnp.int32)

    def _zero(j, carry):
        part0[pl.ds(j * _L, _L)] = zero16
        part1[pl.ds(j * _L, _L)] = zero16
        zbuf[pl.ds(j * _L, _L)] = zero16
        return carry

    lax.fori_loop(0, T // _L, _zero, 0)

    for j in range(16):
        for q in range(8):
            idx2[j, q * _L:(q + 1) * _L] = row * T + j * 128 + q * _L + lanes

    @pl.when(s == 0)
    def _zero_spmem():
        pltpu.sync_copy(zbuf, sp0.at[pl.ds(0, T)])
        pltpu.sync_copy(zbuf, sp0.at[pl.ds(T, T)])
        pltpu.sync_copy(zbuf, sp1.at[pl.ds(0, T)])
        pltpu.sync_copy(zbuf, sp1.at[pl.ds(T, T)])

    for ei in range(2):
        e_cur = ebase + ei
        m0vec = zero16
        for k in range(2):
            lbl = e0l if k == 0 else e1l
            part = part0 if k == 0 else part1

            def _compact(j, cnt, lbl=lbl):
                base = j * _L
                vv = v0l[pl.ds(base, _L)]
                msk = lbl[pl.ds(base, _L)] == e_cur
                pos = cnt + plsc.cumsum(msk.astype(jnp.int32)) - 1
                plsc.store_scatter(segv, [pos], vv, mask=msk)
                plsc.store_scatter(segi, [pos], base + lanes, mask=msk)
                return cnt + plsc.all_reduce_population_count(msk)

            cntv = lax.fori_loop(0, T // _L, _compact, zero16)
            mcount = jnp.max(cntv)
            plsc.store_scatter(segv, [mcount + lanes],
                               jnp.full((_L,), -1.0, jnp.float32))
            plsc.store_scatter(segi, [mcount + lanes],
                               jnp.full((_L,), T, jnp.int32))
            nvec = (mcount + _L - 1) // _L
            addv = m0vec if k == 1 else zero16

            def _rank(mi, carry, part=part, addv=addv, cntv=cntv, nvec=nvec):
                kb = mi * _L
                kv = segv[pl.ds(kb, _L)]
                kt = segi[pl.ds(kb, _L)]

                def _cand(ci, acc):
                    cb = ci * _L
                    for sh in range(_L):
                        perm = cb + ((lanes + sh) & (_L - 1))
                        rv = plsc.load_gather(segv, [perm])
                        rt = plsc.load_gather(segi, [perm])
                        pre = (rv > kv) | ((rv == kv) & (rt < kt))
                        acc = acc + pre.astype(jnp.int32)
                    return acc

                acc = lax.fori_loop(0, nvec, _cand, zero16)
                valid = lanes < (cntv - kb)
                plsc.store_scatter(part, [kt], acc + addv, mask=valid)
                return carry

            lax.fori_loop(0, nvec, _rank, 0)
            if k == 0:
                m0vec = cntv

    plsc.subcore_barrier()
    for j in range(16):
        pltpu.sync_copy(part0.at[pl.ds(j * 128, 128)], sp0.at[idx2.at[j]],
                        add=True)
        pltpu.sync_copy(part1.at[pl.ds(j * 128, 128)], sp1.at[idx2.at[j]],
                        add=True)
    plsc.subcore_barrier()

    off = row * T + tsl
    pltpu.sync_copy(sp0.at[pl.ds(off, _TPW)], pb0)
    pltpu.sync_copy(sp1.at[pl.ds(off, _TPW)], pb1)
    capv = capl[...]
    for j in range(_TPW // _L):
        p0v = pb0[j * _L:(j + 1) * _L]
        p1v = pb1[j * _L:(j + 1) * _L]
        v0v = v0l[pl.ds(tsl + j * _L, _L)]
        v1v = v1s[j * _L:(j + 1) * _L]
        cb0[j * _L:(j + 1) * _L] = jnp.where(p0v < capv, v0v, 0.0)
        cb1[j * _L:(j + 1) * _L] = jnp.where(p1v < capv, v1v, 0.0)
    pltpu.sync_copy(pb0, p0h.at[g, pl.ds(tsl, _TPW)])
    pltpu.sync_copy(pb1, p1h.at[g, pl.ds(tsl, _TPW)])
    pltpu.sync_copy(cb0, cw0h.at[g, pl.ds(tsl, _TPW)])
    pltpu.sync_copy(cb1, cw1h.at[g, pl.ds(tsl, _TPW)])


def _build_sc(G, T, interpret=False):
    mesh = plsc.VectorSubcoreMesh(core_axis_name="c", subcore_axis_name="s",
                                  num_cores=2, num_subcores=16)
    return pl.kernel(
        _sc_router_body,
        out_type=[
            jax.ShapeDtypeStruct((G, T), jnp.int32),
            jax.ShapeDtypeStruct((G, T), jnp.int32),
            jax.ShapeDtypeStruct((G, T), jnp.float32),
            jax.ShapeDtypeStruct((G, T), jnp.float32),
        ],
        mesh=mesh,
        compiler_params=pltpu.CompilerParams(needs_layout_passes=False),
        scratch_types=[
            pltpu.VMEM((T,), jnp.float32),
            pltpu.VMEM((T,), jnp.int32),
            pltpu.VMEM((T,), jnp.int32),
            pltpu.VMEM((_TPW,), jnp.float32),
            pltpu.VMEM((T + _L,), jnp.float32),
            pltpu.VMEM((T + _L,), jnp.int32),
            pltpu.VMEM((T,), jnp.int32),
            pltpu.VMEM((T,), jnp.int32),
            pltpu.VMEM((16, 128), jnp.int32),
            pltpu.VMEM((T,), jnp.int32),
            pltpu.VMEM((_TPW,), jnp.int32),
            pltpu.VMEM((_TPW,), jnp.int32),
            pltpu.VMEM((_TPW,), jnp.float32),
            pltpu.VMEM((_TPW,), jnp.float32),
            pltpu.VMEM((_L,), jnp.int32),
            pltpu.VMEM_SHARED((2 * T,), jnp.int32),
            pltpu.VMEM_SHARED((2 * T,), jnp.int32),
        ],
        interpret=interpret,
    )


def _run(token_inputs, W, b, expert_capacity, interpret=False):
    G, T, D = token_inputs.shape
    E = W.shape[-1]
    v0r, v1r, e0r, e1r, aux, z = _build_dense(G, T, D, E, interpret)(
        token_inputs, W, jnp.reshape(b, (1, E)))
    v0 = v0r.reshape(G, T)
    v1 = v1r.reshape(G, T)
    e0 = e0r.reshape(G, T)
    e1 = e1r.reshape(G, T)
    capv = jnp.full((_L,), expert_capacity, jnp.int32)
    p0, p1, cw0, cw1 = _build_sc(G, T, interpret)(v0, v1, e0, e1, capv)
    disp = jnp.stack([e0, p0, e1, p1], axis=-1).reshape(G, T, 2, 2)
    comb = jnp.stack([cw0, cw1], axis=-1)
    return disp, comb, aux[0, 0], z[0, 0]


def kernel(token_inputs, W, b, num_experts, expert_capacity):
    del num_experts
    return _run(token_inputs, W, b, expert_capacity)


# speedup vs baseline: 1.0132x; 1.0132x over previous
import jax
import jax.numpy as jnp
from jax import lax
from jax.experimental import pallas as pl
from jax.experimental.pallas import tpu as pltpu
from jax.experimental.pallas import tpu_sc as plsc

_L = 16
_TPW = 256


def _dense_body(x_ref, w_ref, b_ref, v0_ref, v1_ref, e0_ref, e1_ref,
                aux_ref, z_ref):
    g = pl.program_id(0)
    G = pl.num_programs(0)
    T = x_ref.shape[1]
    E = w_ref.shape[1]

    x = x_ref[0]
    w = w_ref[...]
    logits = jax.lax.dot_general(
        x, w, (((1,), (0,)), ((), ())),
        preferred_element_type=jnp.float32) + b_ref[...]

    m = jnp.max(logits, axis=-1, keepdims=True)
    ex = jnp.exp(logits - m)
    s = jnp.sum(ex, axis=-1, keepdims=True)
    probs = ex / s

    iota_e = jax.lax.broadcasted_iota(jnp.int32, (T, E), 1)
    v0 = jnp.max(probs, axis=-1, keepdims=True)
    e0 = jnp.min(jnp.where(probs == v0, iota_e, E), axis=-1, keepdims=True)
    probs1 = jnp.where(iota_e == e0, -jnp.inf, probs)
    v1 = jnp.max(probs1, axis=-1, keepdims=True)
    e1 = jnp.min(jnp.where(probs1 == v1, iota_e, E), axis=-1, keepdims=True)

    logz = m + jnp.log(s)
    z_part = jnp.sum(logz * logz) / (G * T)
    a0 = (iota_e == e0).astype(jnp.float32)
    a1 = (iota_e == e1).astype(jnp.float32)
    em_mean = jnp.sum(jnp.maximum(a0, a1), axis=0, keepdims=True) / T
    pm_mean = jnp.sum(probs, axis=0, keepdims=True) / T
    aux_part = jnp.sum(em_mean * pm_mean) * (E / G)

    @pl.when(g == 0)
    def _init():
        aux_ref[...] = jnp.zeros_like(aux_ref)
        z_ref[...] = jnp.zeros_like(z_ref)

    aux_ref[...] += jnp.reshape(aux_part, (1, 1))
    z_ref[...] += jnp.reshape(z_part, (1, 1))

    v0_ref[...] = v0[None]
    v1_ref[...] = v1[None]
    e0_ref[...] = e0[None]
    e1_ref[...] = e1[None]


def _build_dense(G, T, D, E, interpret=False):
    return pl.pallas_call(
        _dense_body,
        grid=(G,),
        in_specs=[
            pl.BlockSpec((1, T, D), lambda g: (g, 0, 0)),
            pl.BlockSpec((D, E), lambda g: (0, 0)),
            pl.BlockSpec((1, E), lambda g: (0, 0)),
        ],
        out_specs=[
            pl.BlockSpec((1, T, 1), lambda g: (g, 0, 0)),
            pl.BlockSpec((1, T, 1), lambda g: (g, 0, 0)),
            pl.BlockSpec((1, T, 1), lambda g: (g, 0, 0)),
            pl.BlockSpec((1, T, 1), lambda g: (g, 0, 0)),
            pl.BlockSpec((1, 1), lambda g: (0, 0)),
            pl.BlockSpec((1, 1), lambda g: (0, 0)),
        ],
        out_shape=[
            jax.ShapeDtypeStruct((G, T, 1), jnp.float32),
            jax.ShapeDtypeStruct((G, T, 1), jnp.float32),
            jax.ShapeDtypeStruct((G, T, 1), jnp.int32),
            jax.ShapeDtypeStruct((G, T, 1), jnp.int32),
            jax.ShapeDtypeStruct((1, 1), jnp.float32),
            jax.ShapeDtypeStruct((1, 1), jnp.float32),
        ],
        interpret=interpret,
    )


def _sc_router_body(v0h, v1h, e0h, e1h, caph,
                    p0h, p1h, cw0h, cw1h,
                    v0l, e0l, e1l, v1s, segv, segi, segv1, segi1,
                    part0, part1, idx2, zbuf, pb0, pb1, cb0, cb1, capl,
                    sp0, sp1):
    c = lax.axis_index("c")
    s = lax.axis_index("s")
    T = v0l.shape[0]
    g = 2 * c + s // 8
    row = s // 8
    ebase = (s % 8) * 2
    tsl = (s % 8) * _TPW
    lanes = lax.iota(jnp.int32, _L)

    pltpu.sync_copy(v0h.at[g], v0l)
    pltpu.sync_copy(e0h.at[g], e0l)
    pltpu.sync_copy(e1h.at[g], e1l)
    pltpu.sync_copy(v1h.at[g, pl.ds(tsl, _TPW)], v1s)
    pltpu.sync_copy(caph, capl)

    zero16 = jnp.zeros((_L,), jnp.int32)

    def _zero(j, carry):
        part0[pl.ds(j * _L, _L)] = zero16
        part1[pl.ds(j * _L, _L)] = zero16
        zbuf[pl.ds(j * _L, _L)] = zero16
        return carry

    lax.fori_loop(0, T // _L, _zero, 0)

    for j in range(16):
        for q in range(8):
            idx2[j, q * _L:(q + 1) * _L] = row * T + j * 128 + q * _L + lanes

    @pl.when(s == 0)
    def _zero_spmem():
        pltpu.sync_copy(zbuf, sp0.at[pl.ds(0, T)])
        pltpu.sync_copy(zbuf, sp0.at[pl.ds(T, T)])
        pltpu.sync_copy(zbuf, sp1.at[pl.ds(0, T)])
        pltpu.sync_copy(zbuf, sp1.at[pl.ds(T, T)])

    CH = 8
    for ei in range(2):
        e_cur = ebase + ei

        def _compact(j, cnts):
            c0, c1 = cnts
            base = j * _L
            vv = v0l[pl.ds(base, _L)]
            tok = base + lanes
            m0 = e0l[pl.ds(base, _L)] == e_cur
            m1 = e1l[pl.ds(base, _L)] == e_cur
            q0 = c0 + plsc.cumsum(m0.astype(jnp.int32)) - 1
            q1 = c1 + plsc.cumsum(m1.astype(jnp.int32)) - 1
            plsc.store_scatter(segv, [q0], vv, mask=m0)
            plsc.store_scatter(segi, [q0], tok, mask=m0)
            plsc.store_scatter(segv1, [q1], vv, mask=m1)
            plsc.store_scatter(segi1, [q1], tok, mask=m1)
            return (c0 + plsc.all_reduce_population_count(m0),
                    c1 + plsc.all_reduce_population_count(m1))

        cnt0v, cnt1v = lax.fori_loop(0, T // _L, _compact, (zero16, zero16))

        for k in range(2):
            sv = segv if k == 0 else segv1
            si = segi if k == 0 else segi1
            part = part0 if k == 0 else part1
            cntv = cnt0v if k == 0 else cnt1v
            addv = zero16 if k == 0 else cnt0v
            mcount = jnp.max(cntv)
            nchunk = (mcount + CH * _L - 1) // (CH * _L)

            def _chunk(ch, carry, sv=sv, si=si, part=part, cntv=cntv,
                       addv=addv, mcount=mcount):
                base = ch * (CH * _L)
                kvs = [sv[pl.ds(base + i * _L, _L)] for i in range(CH)]
                kts = [si[pl.ds(base + i * _L, _L)] for i in range(CH)]

                def _cand(j, accs, sv=sv, si=si, kvs=kvs, kts=kts):
                    jv = jnp.full((_L,), j, jnp.int32)
                    bv = plsc.load_gather(sv, [jv])
                    bt = plsc.load_gather(si, [jv])
                    out = []
                    for i in range(CH):
                        pre = (bv > kvs[i]) | ((bv == kvs[i]) & (bt < kts[i]))
                        out.append(accs[i] + pre.astype(jnp.int32))
                    return tuple(out)

                accs = lax.fori_loop(0, mcount, _cand, (zero16,) * CH)
                for i in range(CH):
                    valid = lanes < (cntv - (base + i * _L))
                    plsc.store_scatter(part, [kts[i]], accs[i] + addv,
                                       mask=valid)
                return carry

            lax.fori_loop(0, nchunk, _chunk, 0)

    plsc.subcore_barrier()
    for j in range(16):
        pltpu.sync_copy(part0.at[pl.ds(j * 128, 128)], sp0.at[idx2.at[j]],
                        add=True)
        pltpu.sync_copy(part1.at[pl.ds(j * 128, 128)], sp1.at[idx2.at[j]],
                        add=True)
    plsc.subcore_barrier()

    off = row * T + tsl
    pltpu.sync_copy(sp0.at[pl.ds(off, _TPW)], pb0)
    pltpu.sync_copy(sp1.at[pl.ds(off, _TPW)], pb1)
    capv = capl[...]
    for j in range(_TPW // _L):
        p0v = pb0[j * _L:(j + 1) * _L]
        p1v = pb1[j * _L:(j + 1) * _L]
        v0v = v0l[pl.ds(tsl + j * _L, _L)]
        v1v = v1s[j * _L:(j + 1) * _L]
        cb0[j * _L:(j + 1) * _L] = jnp.where(p0v < capv, v0v, 0.0)
        cb1[j * _L:(j + 1) * _L] = jnp.where(p1v < capv, v1v, 0.0)
    pltpu.sync_copy(pb0, p0h.at[g, pl.ds(tsl, _TPW)])
    pltpu.sync_copy(pb1, p1h.at[g, pl.ds(tsl, _TPW)])
    pltpu.sync_copy(cb0, cw0h.at[g, pl.ds(tsl, _TPW)])
    pltpu.sync_copy(cb1, cw1h.at[g, pl.ds(tsl, _TPW)])


def _build_sc(G, T, interpret=False):
    mesh = plsc.VectorSubcoreMesh(core_axis_name="c", subcore_axis_name="s",
                                  num_cores=2, num_subcores=16)
    return pl.kernel(
        _sc_router_body,
        out_type=[
            jax.ShapeDtypeStruct((G, T), jnp.int32),
            jax.ShapeDtypeStruct((G, T), jnp.int32),
            jax.ShapeDtypeStruct((G, T), jnp.float32),
            jax.ShapeDtypeStruct((G, T), jnp.float32),
        ],
        mesh=mesh,
        compiler_params=pltpu.CompilerParams(needs_layout_passes=False),
        scratch_types=[
            pltpu.VMEM((T,), jnp.float32),
            pltpu.VMEM((T,), jnp.int32),
            pltpu.VMEM((T,), jnp.int32),
            pltpu.VMEM((_TPW,), jnp.float32),
            pltpu.VMEM((T + _L,), jnp.float32),
            pltpu.VMEM((T + _L,), jnp.int32),
            pltpu.VMEM((T + _L,), jnp.float32),
            pltpu.VMEM((T + _L,), jnp.int32),
            pltpu.VMEM((T,), jnp.int32),
            pltpu.VMEM((T,), jnp.int32),
            pltpu.VMEM((16, 128), jnp.int32),
            pltpu.VMEM((T,), jnp.int32),
            pltpu.VMEM((_TPW,), jnp.int32),
            pltpu.VMEM((_TPW,), jnp.int32),
            pltpu.VMEM((_TPW,), jnp.float32),
            pltpu.VMEM((_TPW,), jnp.float32),
            pltpu.VMEM((_L,), jnp.int32),
            pltpu.VMEM_SHARED((2 * T,), jnp.int32),
            pltpu.VMEM_SHARED((2 * T,), jnp.int32),
        ],
        interpret=interpret,
    )


def _run(token_inputs, W, b, expert_capacity, interpret=False):
    G, T, D = token_inputs.shape
    E = W.shape[-1]
    v0r, v1r, e0r, e1r, aux, z = _build_dense(G, T, D, E, interpret)(
        token_inputs, W, jnp.reshape(b, (1, E)))
    v0 = v0r.reshape(G, T)
    v1 = v1r.reshape(G, T)
    e0 = e0r.reshape(G, T)
    e1 = e1r.reshape(G, T)
    capv = jnp.full((_L,), expert_capacity, jnp.int32)
    p0, p1, cw0, cw1 = _build_sc(G, T, interpret)(v0, v1, e0, e1, capv)
    disp = jnp.stack([e0, p0, e1, p1], axis=-1).reshape(G, T, 2, 2)
    comb = jnp.stack([cw0, cw1], axis=-1)
    return disp, comb, aux[0, 0], z[0, 0]


def kernel(token_inputs, W, b, num_experts, expert_capacity):
    del num_experts
    return _run(token_inputs, W, b, expert_capacity)


# speedup vs baseline: 1.3186x; 1.3014x over previous
import jax
import jax.numpy as jnp
from jax import lax
from jax.experimental import pallas as pl
from jax.experimental.pallas import tpu as pltpu
from jax.experimental.pallas import tpu_sc as plsc

_L = 16
_TPW = 256


def _dense_body(x_ref, w_ref, b_ref, v0_ref, v1_ref, e0_ref, e1_ref,
                aux_ref, z_ref):
    g = pl.program_id(0)
    G = pl.num_programs(0)
    T = x_ref.shape[1]
    E = w_ref.shape[1]

    x = x_ref[0]
    w = w_ref[...]
    logits = jax.lax.dot_general(
        x, w, (((1,), (0,)), ((), ())),
        preferred_element_type=jnp.float32) + b_ref[...]
    lt = jnp.transpose(logits)

    m = jnp.max(lt, axis=0, keepdims=True)
    ex = jnp.exp(lt - m)
    s = jnp.sum(ex, axis=0, keepdims=True)
    probs = ex / s

    iota_e = jax.lax.broadcasted_iota(jnp.int32, (E, T), 0)
    v0 = jnp.max(probs, axis=0, keepdims=True)
    e0 = jnp.min(jnp.where(probs == v0, iota_e, E), axis=0, keepdims=True)
    probs1 = jnp.where(iota_e == e0, -jnp.inf, probs)
    v1 = jnp.max(probs1, axis=0, keepdims=True)
    e1 = jnp.min(jnp.where(probs1 == v1, iota_e, E), axis=0, keepdims=True)

    logz = m + jnp.log(s)
    z_part = jnp.sum(logz * logz) / (G * T)
    a0 = (iota_e == e0).astype(jnp.float32)
    a1 = (iota_e == e1).astype(jnp.float32)
    em_mean = jnp.sum(jnp.maximum(a0, a1), axis=1, keepdims=True) / T
    pm_mean = jnp.sum(probs, axis=1, keepdims=True) / T
    aux_part = jnp.sum(em_mean * pm_mean) * (E / G)

    @pl.when(g == 0)
    def _init():
        aux_ref[...] = jnp.zeros_like(aux_ref)
        z_ref[...] = jnp.zeros_like(z_ref)

    aux_ref[...] += jnp.reshape(aux_part, (1, 1))
    z_ref[...] += jnp.reshape(z_part, (1, 1))

    v0_ref[...] = v0[None]
    v1_ref[...] = v1[None]
    e0_ref[...] = e0[None]
    e1_ref[...] = e1[None]


def _build_dense(G, T, D, E, interpret=False):
    return pl.pallas_call(
        _dense_body,
        grid=(G,),
        in_specs=[
            pl.BlockSpec((1, T, D), lambda g: (g, 0, 0)),
            pl.BlockSpec((D, E), lambda g: (0, 0)),
            pl.BlockSpec((1, E), lambda g: (0, 0)),
        ],
        out_specs=[
            pl.BlockSpec((1, 1, T), lambda g: (g, 0, 0)),
            pl.BlockSpec((1, 1, T), lambda g: (g, 0, 0)),
            pl.BlockSpec((1, 1, T), lambda g: (g, 0, 0)),
            pl.BlockSpec((1, 1, T), lambda g: (g, 0, 0)),
            pl.BlockSpec((1, 1), lambda g: (0, 0)),
            pl.BlockSpec((1, 1), lambda g: (0, 0)),
        ],
        out_shape=[
            jax.ShapeDtypeStruct((G, 1, T), jnp.float32),
            jax.ShapeDtypeStruct((G, 1, T), jnp.float32),
            jax.ShapeDtypeStruct((G, 1, T), jnp.int32),
            jax.ShapeDtypeStruct((G, 1, T), jnp.int32),
            jax.ShapeDtypeStruct((1, 1), jnp.float32),
            jax.ShapeDtypeStruct((1, 1), jnp.float32),
        ],
        interpret=interpret,
    )


def _sc_router_body(v0h, v1h, e0h, e1h, caph,
                    p0h, p1h, cw0h, cw1h,
                    v0l, e0l, e1l, v1s, segv, segi, segv1, segi1,
                    part0, part1, idx2, zbuf, pb0, pb1, cb0, cb1, capl,
                    sp0, sp1):
    c = lax.axis_index("c")
    s = lax.axis_index("s")
    T = v0l.shape[0]
    g = 2 * c + s // 8
    row = s // 8
    ebase = (s % 8) * 2
    tsl = (s % 8) * _TPW
    lanes = lax.iota(jnp.int32, _L)

    pltpu.sync_copy(v0h.at[g], v0l)
    pltpu.sync_copy(e0h.at[g], e0l)
    pltpu.sync_copy(e1h.at[g], e1l)
    pltpu.sync_copy(v1h.at[g, pl.ds(tsl, _TPW)], v1s)
    pltpu.sync_copy(caph, capl)

    zero16 = jnp.zeros((_L,), jnp.int32)

    def _zero(j, carry):
        part0[pl.ds(j * _L, _L)] = zero16
        part1[pl.ds(j * _L, _L)] = zero16
        zbuf[pl.ds(j * _L, _L)] = zero16
        return carry

    lax.fori_loop(0, T // _L, _zero, 0)

    for j in range(16):
        for q in range(8):
            idx2[j, q * _L:(q + 1) * _L] = row * T + j * 128 + q * _L + lanes

    @pl.when(s == 0)
    def _zero_spmem():
        pltpu.sync_copy(zbuf, sp0.at[pl.ds(0, T)])
        pltpu.sync_copy(zbuf, sp0.at[pl.ds(T, T)])
        pltpu.sync_copy(zbuf, sp1.at[pl.ds(0, T)])
        pltpu.sync_copy(zbuf, sp1.at[pl.ds(T, T)])

    CH = 8
    for ei in range(2):
        e_cur = ebase + ei

        def _compact(j, cnts):
            c0, c1 = cnts
            base = j * _L
            vv = v0l[pl.ds(base, _L)]
            tok = base + lanes
            m0 = e0l[pl.ds(base, _L)] == e_cur
            m1 = e1l[pl.ds(base, _L)] == e_cur
            q0 = c0 + plsc.cumsum(m0.astype(jnp.int32)) - 1
            q1 = c1 + plsc.cumsum(m1.astype(jnp.int32)) - 1
            plsc.store_scatter(segv, [q0], vv, mask=m0)
            plsc.store_scatter(segi, [q0], tok, mask=m0)
            plsc.store_scatter(segv1, [q1], vv, mask=m1)
            plsc.store_scatter(segi1, [q1], tok, mask=m1)
            return (c0 + plsc.all_reduce_population_count(m0),
                    c1 + plsc.all_reduce_population_count(m1))

        cnt0v, cnt1v = lax.fori_loop(0, T // _L, _compact, (zero16, zero16))

        for k in range(2):
            sv = segv if k == 0 else segv1
            si = segi if k == 0 else segi1
            part = part0 if k == 0 else part1
            cntv = cnt0v if k == 0 else cnt1v
            addv = zero16 if k == 0 else cnt0v
            mcount = jnp.max(cntv)
            nchunk = (mcount + CH * _L - 1) // (CH * _L)

            def _chunk(ch, carry, sv=sv, si=si, part=part, cntv=cntv,
                       addv=addv, mcount=mcount):
                base = ch * (CH * _L)
                kvs = [sv[pl.ds(base + i * _L, _L)] for i in range(CH)]
                kts = [si[pl.ds(base + i * _L, _L)] for i in range(CH)]

                def _cand(j, accs, sv=sv, si=si, kvs=kvs, kts=kts):
                    jv = jnp.full((_L,), j, jnp.int32)
                    bv = plsc.load_gather(sv, [jv])
                    bt = plsc.load_gather(si, [jv])
                    out = []
                    for i in range(CH):
                        pre = (bv > kvs[i]) | ((bv == kvs[i]) & (bt < kts[i]))
                        out.append(accs[i] + pre.astype(jnp.int32))
                    return tuple(out)

                accs = lax.fori_loop(0, mcount, _cand, (zero16,) * CH)
                for i in range(CH):
                    valid = lanes < (cntv - (base + i * _L))
                    plsc.store_scatter(part, [kts[i]], accs[i] + addv,
                                       mask=valid)
                return carry

            lax.fori_loop(0, nchunk, _chunk, 0)

    plsc.subcore_barrier()
    for j in range(16):
        pltpu.sync_copy(part0.at[pl.ds(j * 128, 128)], sp0.at[idx2.at[j]],
                        add=True)
        pltpu.sync_copy(part1.at[pl.ds(j * 128, 128)], sp1.at[idx2.at[j]],
                        add=True)
    plsc.subcore_barrier()

    off = row * T + tsl
    pltpu.sync_copy(sp0.at[pl.ds(off, _TPW)], pb0)
    pltpu.sync_copy(sp1.at[pl.ds(off, _TPW)], pb1)
    capv = capl[...]
    for j in range(_TPW // _L):
        p0v = pb0[j * _L:(j + 1) * _L]
        p1v = pb1[j * _L:(j + 1) * _L]
        v0v = v0l[pl.ds(tsl + j * _L, _L)]
        v1v = v1s[j * _L:(j + 1) * _L]
        cb0[j * _L:(j + 1) * _L] = jnp.where(p0v < capv, v0v, 0.0)
        cb1[j * _L:(j + 1) * _L] = jnp.where(p1v < capv, v1v, 0.0)
    pltpu.sync_copy(pb0, p0h.at[g, pl.ds(tsl, _TPW)])
    pltpu.sync_copy(pb1, p1h.at[g, pl.ds(tsl, _TPW)])
    pltpu.sync_copy(cb0, cw0h.at[g, pl.ds(tsl, _TPW)])
    pltpu.sync_copy(cb1, cw1h.at[g, pl.ds(tsl, _TPW)])


def _build_sc(G, T, interpret=False):
    mesh = plsc.VectorSubcoreMesh(core_axis_name="c", subcore_axis_name="s",
                                  num_cores=2, num_subcores=16)
    return pl.kernel(
        _sc_router_body,
        out_type=[
            jax.ShapeDtypeStruct((G, T), jnp.int32),
            jax.ShapeDtypeStruct((G, T), jnp.int32),
            jax.ShapeDtypeStruct((G, T), jnp.float32),
            jax.ShapeDtypeStruct((G, T), jnp.float32),
        ],
        mesh=mesh,
        compiler_params=pltpu.CompilerParams(needs_layout_passes=False),
        scratch_types=[
            pltpu.VMEM((T,), jnp.float32),
            pltpu.VMEM((T,), jnp.int32),
            pltpu.VMEM((T,), jnp.int32),
            pltpu.VMEM((_TPW,), jnp.float32),
            pltpu.VMEM((T + _L,), jnp.float32),
            pltpu.VMEM((T + _L,), jnp.int32),
            pltpu.VMEM((T + _L,), jnp.float32),
            pltpu.VMEM((T + _L,), jnp.int32),
            pltpu.VMEM((T,), jnp.int32),
            pltpu.VMEM((T,), jnp.int32),
            pltpu.VMEM((16, 128), jnp.int32),
            pltpu.VMEM((T,), jnp.int32),
            pltpu.VMEM((_TPW,), jnp.int32),
            pltpu.VMEM((_TPW,), jnp.int32),
            pltpu.VMEM((_TPW,), jnp.float32),
            pltpu.VMEM((_TPW,), jnp.float32),
            pltpu.VMEM((_L,), jnp.int32),
            pltpu.VMEM_SHARED((2 * T,), jnp.int32),
            pltpu.VMEM_SHARED((2 * T,), jnp.int32),
        ],
        interpret=interpret,
    )


def _run(token_inputs, W, b, expert_capacity, interpret=False):
    G, T, D = token_inputs.shape
    E = W.shape[-1]
    v0r, v1r, e0r, e1r, aux, z = _build_dense(G, T, D, E, interpret)(
        token_inputs, W, jnp.reshape(b, (1, E)))
    v0 = v0r.reshape(G, T)
    v1 = v1r.reshape(G, T)
    e0 = e0r.reshape(G, T)
    e1 = e1r.reshape(G, T)
    capv = jnp.full((_L,), expert_capacity, jnp.int32)
    p0, p1, cw0, cw1 = _build_sc(G, T, interpret)(v0, v1, e0, e1, capv)
    disp = jnp.stack([e0, p0, e1, p1], axis=-1).reshape(G, T, 2, 2)
    comb = jnp.stack([cw0, cw1], axis=-1)
    return disp, comb, aux[0, 0], z[0, 0]


def kernel(token_inputs, W, b, num_experts, expert_capacity):
    del num_experts
    return _run(token_inputs, W, b, expert_capacity)
